# baseline (speedup 1.0000x reference)
import functools

import jax
import jax.numpy as jnp
from jax import lax
from jax.experimental import pallas as pl
from jax.experimental.pallas import tpu as pltpu
from jax.experimental.pallas import tpu_sc as plsc

_NUM_CORES = 2
_NUM_SUBCORES = 16
_NUM_WORKERS = _NUM_CORES * _NUM_SUBCORES
_LANES = 16

_B = 4096
_L = 20
_D = 128
_DV = _D // _LANES

_SENT_PER_W = _B // _NUM_WORKERS
_CH = 4
_ROWS = _CH * _L
_NCHUNK = _SENT_PER_W // _CH
_NBUF = 4


def _body(table_hbm, sp_hbm, wi_hbm, ri_hbm, neg_hbm, out_hbm,
          idx_v, ri_v, neg_v, sp_v, rows0, rows1, rows2, rows3, out_v,
          sem0, sem1, sem2, sem3):
  rows_bufs = (rows0, rows1, rows2, rows3)
  sems = (sem0, sem1, sem2, sem3)
  wid = lax.axis_index("s") * _NUM_CORES + lax.axis_index("c")
  sbase = pl.multiple_of(wid * _SENT_PER_W, 8)
  tbase = pl.multiple_of(wid * (_SENT_PER_W * _L), 8)

  pltpu.sync_copy(sp_hbm, sp_v)
  pltpu.sync_copy(wi_hbm.at[pl.ds(tbase, _SENT_PER_W * _L)], idx_v)
  pltpu.sync_copy(ri_hbm.at[pl.ds(tbase, _SENT_PER_W * _L)],
                  ri_v.at[pl.ds(0, _SENT_PER_W * _L)])
  pltpu.sync_copy(neg_hbm.at[pl.ds(sbase, _SENT_PER_W)],
                  neg_v.at[pl.ds(0, _SENT_PER_W)])

  def start_gather(c, rows, sem):
    off = pl.multiple_of(c * _ROWS, 8)
    pltpu.async_copy(table_hbm.at[idx_v.at[pl.ds(off, _ROWS)]], rows, sem)

  def wait_gather(rows, sem):
    pltpu.make_async_copy(
        table_hbm.at[idx_v.at[pl.ds(0, _ROWS)]], rows, sem).wait()

  def process(c, rows):
    sp_vals = sp_v[pl.ds(0, _LANES)]

    def pair_body(pr, _):
      tok0 = pl.multiple_of((c * _CH + 2 * pr) * _L, 8)
      rv = [ri_v[pl.ds(tok0 + v * _LANES, _LANES)] for v in range(3)]
      wv = [sp_vals.at[r].get(mode="promise_in_bounds") for r in rv]
      wls_pair = (
          [wv[0][l] for l in range(_LANES)] + [wv[1][l] for l in range(4)],
          [wv[1][l] for l in range(4, _LANES)] + [wv[2][l] for l in range(8)],
      )
      for s in range(2):
        sent = c * _CH + 2 * pr + s
        rbase = (2 * pr + s) * _L
        wls = wls_pair[s]
        tot = wls[0]
        for l in range(1, _L):
          tot = tot + wls[l]
        inv = (jnp.ones((_LANES,), jnp.float32)
               / jnp.broadcast_to(tot, (_LANES,)))
        nb = neg_v[pl.ds(sent, _LANES)][0]
        inv_lo = jnp.where(nb != 0, -inv, inv)
        accs = [jnp.zeros((_LANES,), jnp.float32) for _ in range(_DV)]
        for l in range(_L):
          w = wls[l]
          for j in range(_DV):
            accs[j] = accs[j] + w * rows[rbase + l, pl.ds(j * _LANES, _LANES)]
        for j in range(_DV):
          scale = inv_lo if j < _DV // 2 else inv
          out_v[sent, pl.ds(j * _LANES, _LANES)] = accs[j] * scale
      return 0

    lax.fori_loop(0, _CH // 2, pair_body, 0)

  bufs = list(zip(rows_bufs, sems))
  for b, (rows, sem) in enumerate(bufs):
    start_gather(b, rows, sem)

  def chunk_group(i, _):
    c = _NBUF * i
    for b, (rows, sem) in enumerate(bufs):
      wait_gather(rows, sem)
      process(c + b, rows)

      @pl.when(c + b + _NBUF < _NCHUNK)
      def _():
        start_gather(c + b + _NBUF, rows, sem)
    return 0

  lax.fori_loop(0, _NCHUNK // _NBUF, chunk_group, 0)

  pltpu.sync_copy(out_v, out_hbm.at[pl.ds(sbase, _SENT_PER_W)])


@jax.jit
def _run(table, sp_pad, wi_flat, ri_flat, neg_i32):
  mesh = plsc.VectorSubcoreMesh(
      core_axis_name="c", subcore_axis_name="s",
      num_cores=_NUM_CORES, num_subcores=_NUM_SUBCORES)
  fn = pl.kernel(
      _body,
      out_type=jax.ShapeDtypeStruct((_B, _D), jnp.float32),
      mesh=mesh,
      scratch_types=[
          pltpu.VMEM((_SENT_PER_W * _L,), jnp.int32),
          pltpu.VMEM((_SENT_PER_W * _L + _LANES,), jnp.int32),
          pltpu.VMEM((_SENT_PER_W + _LANES,), jnp.int32),
          pltpu.VMEM((_LANES,), jnp.float32),
          pltpu.VMEM((_ROWS, _D), jnp.float32),
          pltpu.VMEM((_ROWS, _D), jnp.float32),
          pltpu.VMEM((_ROWS, _D), jnp.float32),
          pltpu.VMEM((_ROWS, _D), jnp.float32),
          pltpu.VMEM((_SENT_PER_W, _D), jnp.float32),
          pltpu.SemaphoreType.DMA,
          pltpu.SemaphoreType.DMA,
          pltpu.SemaphoreType.DMA,
          pltpu.SemaphoreType.DMA,
      ],
  )
  return fn(table, sp_pad, wi_flat, ri_flat, neg_i32)


def kernel(root_embeddings, role_weights, word_indices, role_indices, negita):
  sp = jax.nn.softplus(role_weights.astype(jnp.float32))
  sp_pad = jnp.pad(sp, (0, _LANES - sp.shape[0]))
  wi_flat = word_indices.astype(jnp.int32).reshape(-1)
  ri_flat = role_indices.astype(jnp.int32).reshape(-1)
  neg_i32 = negita.astype(jnp.int32)
  return _run(root_embeddings, sp_pad, wi_flat, ri_flat, neg_i32)


# speedup vs baseline: 1.0960x; 1.0960x over previous
import functools

import jax
import jax.numpy as jnp
from jax import lax
from jax.experimental import pallas as pl
from jax.experimental.pallas import tpu as pltpu
from jax.experimental.pallas import tpu_sc as plsc

_NUM_CORES = 2
_NUM_SUBCORES = 16
_NUM_WORKERS = _NUM_CORES * _NUM_SUBCORES
_LANES = 16

_B = 4096
_L = 20
_LP = 32
_D = 128
_DV = _D // _LANES

_SENT_PER_W = _B // _NUM_WORKERS
_CH = 4
_ROWS = _CH * _L
_NCHUNK = _SENT_PER_W // _CH
_NBUF = 4


def _body(table_hbm, sp_hbm, wi_hbm, ri_hbm, neg_hbm, out_hbm,
          idx_v, ri_v, neg_v, sp_v, rows0, rows1, rows2, rows3, out_v,
          sem0, sem1, sem2, sem3):
  rows_bufs = (rows0, rows1, rows2, rows3)
  sems = (sem0, sem1, sem2, sem3)
  wid = lax.axis_index("s") * _NUM_CORES + lax.axis_index("c")
  sbase = pl.multiple_of(wid * _SENT_PER_W, 8)
  tbase = pl.multiple_of(wid * (_SENT_PER_W * _L), 8)
  rbase_w = pl.multiple_of(wid * (_SENT_PER_W * _LP), 8)

  pltpu.sync_copy(sp_hbm, sp_v)
  pltpu.sync_copy(wi_hbm.at[pl.ds(tbase, _SENT_PER_W * _L)], idx_v)
  pltpu.sync_copy(ri_hbm.at[pl.ds(rbase_w, _SENT_PER_W * _LP)], ri_v)
  pltpu.sync_copy(neg_hbm.at[pl.ds(sbase, _SENT_PER_W)],
                  neg_v.at[pl.ds(0, _SENT_PER_W)])

  def start_gather(c, rows, sem):
    off = pl.multiple_of(c * _ROWS, 8)
    pltpu.async_copy(table_hbm.at[idx_v.at[pl.ds(off, _ROWS)]], rows, sem)

  def wait_gather(rows, sem):
    pltpu.make_async_copy(
        table_hbm.at[idx_v.at[pl.ds(0, _ROWS)]], rows, sem).wait()

  lanes = lax.iota(jnp.int32, _LANES)

  def lane_sum(v):
    for step in (1, 2, 4, 8):
      v = v + v.at[lanes ^ step].get(mode="promise_in_bounds")
    return v

  def process(c, rows):
    def sent_body(s, _):
      sent = c * _CH + s
      rbase = s * _L
      r0 = ri_v[pl.ds(sent * _LP, _LANES)]
      r1 = ri_v[pl.ds(sent * _LP + _LANES, _LANES)]
      sp_vals = sp_v[pl.ds(0, _LANES)]
      w0 = sp_vals.at[r0].get(mode="promise_in_bounds")
      w1 = sp_vals.at[r1].get(mode="promise_in_bounds")
      totv = lane_sum(w0 + w1)
      inv = jnp.ones((_LANES,), jnp.float32) / totv
      nb = neg_v[pl.ds(sent, _LANES)][0]
      inv_lo = jnp.where(nb != 0, -inv, inv)
      wls = [w0[l] for l in range(_LANES)] + [w1[l] for l in range(_L - _LANES)]
      accs = [jnp.zeros((_LANES,), jnp.float32) for _ in range(_DV)]
      for l in range(_L):
        w = wls[l]
        for j in range(_DV):
          accs[j] = accs[j] + w * rows[rbase + l, pl.ds(j * _LANES, _LANES)]
      for j in range(_DV):
        scale = inv_lo if j < _DV // 2 else inv
        out_v[sent, pl.ds(j * _LANES, _LANES)] = accs[j] * scale
      return 0

    lax.fori_loop(0, _CH, sent_body, 0)

  bufs = list(zip(rows_bufs, sems))
  for b, (rows, sem) in enumerate(bufs):
    start_gather(b, rows, sem)

  def chunk_group(i, _):
    c = _NBUF * i
    for b, (rows, sem) in enumerate(bufs):
      wait_gather(rows, sem)
      process(c + b, rows)

      @pl.when(c + b + _NBUF < _NCHUNK)
      def _():
        start_gather(c + b + _NBUF, rows, sem)
    return 0

  lax.fori_loop(0, _NCHUNK // _NBUF, chunk_group, 0)

  pltpu.sync_copy(out_v, out_hbm.at[pl.ds(sbase, _SENT_PER_W)])


@jax.jit
def _run(table, sp_pad, wi_flat, ri_flat, neg_i32):
  mesh = plsc.VectorSubcoreMesh(
      core_axis_name="c", subcore_axis_name="s",
      num_cores=_NUM_CORES, num_subcores=_NUM_SUBCORES)
  fn = pl.kernel(
      _body,
      out_type=jax.ShapeDtypeStruct((_B, _D), jnp.float32),
      mesh=mesh,
      scratch_types=[
          pltpu.VMEM((_SENT_PER_W * _L,), jnp.int32),
          pltpu.VMEM((_SENT_PER_W * _LP,), jnp.int32),
          pltpu.VMEM((_SENT_PER_W + _LANES,), jnp.int32),
          pltpu.VMEM((_LANES,), jnp.float32),
          pltpu.VMEM((_ROWS, _D), jnp.float32),
          pltpu.VMEM((_ROWS, _D), jnp.float32),
          pltpu.VMEM((_ROWS, _D), jnp.float32),
          pltpu.VMEM((_ROWS, _D), jnp.float32),
          pltpu.VMEM((_SENT_PER_W, _D), jnp.float32),
          pltpu.SemaphoreType.DMA,
          pltpu.SemaphoreType.DMA,
          pltpu.SemaphoreType.DMA,
          pltpu.SemaphoreType.DMA,
      ],
  )
  return fn(table, sp_pad, wi_flat, ri_flat, neg_i32)


def kernel(root_embeddings, role_weights, word_indices, role_indices, negita):
  sp = jax.nn.softplus(role_weights.astype(jnp.float32))
  sp_pad = jnp.pad(sp, (0, _LANES - sp.shape[0]))
  wi_flat = word_indices.astype(jnp.int32).reshape(-1)
  ri_pad = jnp.pad(role_indices.astype(jnp.int32),
                   ((0, 0), (0, _LP - _L)), constant_values=sp.shape[0])
  neg_i32 = negita.astype(jnp.int32)
  return _run(root_embeddings, sp_pad, wi_flat, ri_pad.reshape(-1), neg_i32)
